# baseline (speedup 1.0000x reference)
import functools

import jax
import jax.numpy as jnp
from jax import lax
from jax.experimental import pallas as pl
from jax.experimental.pallas import tpu as pltpu
from jax.experimental.pallas import tpu_sc as plsc

N = 10000
E = 160000
F_IN = 256
H = 512
C_OUT = 64
OUT = 32
G = 64

NUM_CORES = 2
NUM_SUBCORES = 16

FC = 128
K_EDGE = 128
EPT = 10240
E_PAD = NUM_SUBCORES * EPT
NB = EPT // K_EDGE
GB = 20
NBG = NB // GB
ACC_ROWS = N + 8
RPT_A = 640
RPT_LAST = N - (NUM_SUBCORES - 1) * RPT_A

BR = 1000


def _make_sc_agg(num_chunks):
    ppc = num_chunks // NUM_CORES
    mesh = plsc.VectorSubcoreMesh(core_axis_name="c", subcore_axis_name="s")

    @functools.partial(
        pl.kernel,
        mesh=mesh,
        out_type=[jax.ShapeDtypeStruct((N, FC), jnp.float32)
                  for _ in range(num_chunks)],
        scratch_types=[
            pltpu.VMEM((GB, K_EDGE), jnp.int32),
            pltpu.VMEM((GB, K_EDGE), jnp.int32),
            pltpu.VMEM((K_EDGE, FC), jnp.float32),
            pltpu.VMEM((K_EDGE, FC), jnp.float32),
            pltpu.VMEM_SHARED((ACC_ROWS, FC), jnp.float32),
            pltpu.SemaphoreType.DMA,
            pltpu.SemaphoreType.DMA,
        ],
    )
    def agg_kernel(*refs):
        hs = refs[:num_chunks]
        src_hbm = refs[num_chunks]
        dst_hbm = refs[num_chunks + 1]
        zeros_hbm = refs[num_chunks + 2]
        outs = refs[num_chunks + 3:2 * num_chunks + 3]
        src_g, dst_g, rows0, rows1, acc, sem0, sem1 = \
            refs[2 * num_chunks + 3:]

        core = lax.axis_index("c")
        sub = lax.axis_index("s")
        r0 = sub * RPT_A

        def rows_copy(src_ref, dst_ref):
            @pl.when(sub < NUM_SUBCORES - 1)
            def _():
                pltpu.sync_copy(src_ref.at[pl.ds(r0, RPT_A)],
                                dst_ref.at[pl.ds(r0, RPT_A)])

            @pl.when(sub == NUM_SUBCORES - 1)
            def _():
                pltpu.sync_copy(src_ref.at[pl.ds(r0, RPT_LAST)],
                                dst_ref.at[pl.ds(r0, RPT_LAST)])

        def edge_loop(h_hbm):
            def g_copy(j, rows, sem):
                return pltpu.make_async_copy(h_hbm.at[src_g.at[j]],
                                             rows, sem)

            def scatter(j, rows):
                pltpu.sync_copy(rows, acc.at[dst_g.at[j]], add=True)

            for g in range(NBG):
                pltpu.sync_copy(src_hbm.at[sub, g], src_g)
                pltpu.sync_copy(dst_hbm.at[sub, g], dst_g)
                g_copy(0, rows0, sem0).start()

                def pair(jo, carry):
                    j0 = 2 * jo
                    g_copy(j0 + 1, rows1, sem1).start()
                    g_copy(j0, rows0, sem0).wait()
                    scatter(j0, rows0)
                    g_copy(j0 + 2, rows0, sem0).start()
                    g_copy(j0 + 1, rows1, sem1).wait()
                    scatter(j0 + 1, rows1)
                    return carry

                if GB % 2:
                    lax.fori_loop(0, (GB - 1) // 2, pair, 0)
                    g_copy(GB - 1, rows0, sem0).wait()
                    scatter(GB - 1, rows0)
                else:
                    lax.fori_loop(0, GB // 2 - 1, pair, 0)
                    g_copy(GB - 1, rows1, sem1).start()
                    g_copy(GB - 2, rows0, sem0).wait()
                    scatter(GB - 2, rows0)
                    g_copy(GB - 1, rows1, sem1).wait()
                    scatter(GB - 1, rows1)

        for p in range(ppc):
            rows_copy(zeros_hbm, acc)
            plsc.subcore_barrier()

            @pl.when(core == 0)
            def _():
                edge_loop(hs[p])

            @pl.when(core == 1)
            def _():
                edge_loop(hs[ppc + p])

            plsc.subcore_barrier()

            @pl.when(core == 0)
            def _():
                rows_copy(acc, outs[p])

            @pl.when(core == 1)
            def _():
                rows_copy(acc, outs[ppc + p])

            plsc.subcore_barrier()

    return agg_kernel


def _make_tc_layer(ch_in, relu_out):
    fi = ch_in * FC
    n_out = H // FC

    def body(*refs):
        hs = refs[:ch_in]
        ags = refs[ch_in:2 * ch_in]
        w1, b1, w2, b2 = refs[2 * ch_in:2 * ch_in + 4]
        outs = refs[2 * ch_in + 4:]
        acc = None
        for c in range(ch_in):
            xin = hs[c][...] + ags[c][...]
            part = jnp.dot(xin, w1[c * FC:(c + 1) * FC, :],
                           preferred_element_type=jnp.float32)
            acc = part if acc is None else acc + part
        m = jnp.maximum(acc + b1[...], 0.0)
        o = jnp.dot(m, w2[...], preferred_element_type=jnp.float32) + b2[...]
        if relu_out:
            o = jnp.maximum(o, 0.0)
        for c in range(n_out):
            outs[c][...] = o[:, c * FC:(c + 1) * FC]

    row_spec = pl.BlockSpec((BR, FC), lambda i: (i, 0))
    in_specs = ([row_spec] * (2 * ch_in) + [
        pl.BlockSpec((fi, H), lambda i: (0, 0)),
        pl.BlockSpec((1, H), lambda i: (0, 0)),
        pl.BlockSpec((H, H), lambda i: (0, 0)),
        pl.BlockSpec((1, H), lambda i: (0, 0)),
    ])
    return pl.pallas_call(
        body,
        grid=(N // BR,),
        in_specs=in_specs,
        out_specs=[row_spec] * n_out,
        out_shape=[jax.ShapeDtypeStruct((N, FC), jnp.float32)
                   for _ in range(n_out)],
    )


def _make_tc_layer_final(ch_in):
    fi = ch_in * FC

    def body(*refs):
        hs = refs[:ch_in]
        ags = refs[ch_in:2 * ch_in]
        (w1, b1, w2, b2, ncw, ncb, batch_r, batch_c,
         node_ref, sums_ref, maxes_ref, counts_ref) = refs[2 * ch_in:]

        acc = None
        for c in range(ch_in):
            xin = hs[c][...] + ags[c][...]
            part = jnp.dot(xin, w1[c * FC:(c + 1) * FC, :],
                           preferred_element_type=jnp.float32)
            acc = part if acc is None else acc + part
        m = jnp.maximum(acc + b1[...], 0.0)
        o = jnp.dot(m, w2[...], preferred_element_type=jnp.float32) + b2[...]

        node_ref[...] = (jnp.dot(jnp.maximum(o, 0.0), ncw[...],
                                 preferred_element_type=jnp.float32)
                         + ncb[...])

        i = pl.program_id(0)

        @pl.when(i == 0)
        def _():
            sums_ref[...] = jnp.zeros_like(sums_ref)
            counts_ref[...] = jnp.zeros_like(counts_ref)
            maxes_ref[...] = jnp.full_like(maxes_ref, -1e30)

        ids_row = batch_r[0]
        ids_col = batch_c[0]
        onehot = (lax.broadcasted_iota(jnp.int32, (G, BR), 0)
                  == ids_row).astype(jnp.float32)
        sums_ref[...] += jnp.dot(onehot, o,
                                 preferred_element_type=jnp.float32)
        cnt = jnp.sum(onehot, axis=1, keepdims=True)
        counts_ref[...] += jnp.broadcast_to(cnt, counts_ref.shape)

        def mbody(g, cur):
            msk = ids_col == g
            mx = jnp.max(jnp.where(msk, o, -1e30), axis=0, keepdims=True)
            sel = lax.broadcasted_iota(jnp.int32, (G, 1), 0) == g
            return jnp.where(sel, jnp.maximum(cur, mx), cur)

        maxes_ref[...] = lax.fori_loop(ids_col[0, 0], ids_col[BR - 1, 0] + 1,
                                       mbody, maxes_ref[...])

    row_spec = pl.BlockSpec((BR, FC), lambda i: (i, 0))
    full = lambda shape: pl.BlockSpec(shape, lambda i: tuple(0 for _ in shape))
    in_specs = ([row_spec] * (2 * ch_in) + [
        full((fi, H)),
        full((1, H)),
        full((H, H)),
        full((1, H)),
        full((H, C_OUT)),
        full((1, C_OUT)),
        pl.BlockSpec((1, 1, BR), lambda i: (i, 0, 0)),
        pl.BlockSpec((1, BR, 1), lambda i: (i, 0, 0)),
    ])
    out_specs = [
        pl.BlockSpec((BR, C_OUT), lambda i: (i, 0)),
        full((G, H)),
        full((G, H)),
        full((G, FC)),
    ]
    out_shape = [
        jax.ShapeDtypeStruct((N, C_OUT), jnp.float32),
        jax.ShapeDtypeStruct((G, H), jnp.float32),
        jax.ShapeDtypeStruct((G, H), jnp.float32),
        jax.ShapeDtypeStruct((G, FC), jnp.float32),
    ]
    return pl.pallas_call(
        body,
        grid=(N // BR,),
        in_specs=in_specs,
        out_specs=out_specs,
        out_shape=out_shape,
    )


def _head_body(sums_ref, maxes_ref, counts_ref, p1w, p1b, p2w, p2b, out_ref):
    cnt = counts_ref[...][:, 0:1]
    mean = sums_ref[...] / jnp.maximum(cnt, 1.0)
    mx = jnp.where(cnt > 0.0, maxes_ref[...], 0.0)
    gcat = jnp.concatenate([mean, mx], axis=1)
    p = jnp.maximum(jnp.dot(gcat, p1w[...],
                            preferred_element_type=jnp.float32) + p1b[...],
                    0.0)
    out_ref[...] = (jnp.dot(p, p2w[...],
                            preferred_element_type=jnp.float32) + p2b[...])


_head_call = pl.pallas_call(
    _head_body,
    out_shape=jax.ShapeDtypeStruct((G, OUT), jnp.float32),
)


def kernel(x, edge_index, batch, params):
    pad = E_PAD - E
    src = jnp.concatenate(
        [edge_index[0], jnp.zeros((pad,), jnp.int32)]
    ).reshape(NUM_SUBCORES, NBG, GB, K_EDGE)
    dst = jnp.concatenate(
        [edge_index[1], jnp.full((pad,), N, jnp.int32)]
    ).reshape(NUM_SUBCORES, NBG, GB, K_EDGE)
    zeros = jnp.zeros((N, FC), jnp.float32)
    batch_r = batch.reshape(N // BR, 1, BR)
    batch_c = batch.reshape(N // BR, BR, 1)

    agg2 = _make_sc_agg(2)
    agg4 = _make_sc_agg(4)

    h = [x[:, c * FC:(c + 1) * FC] for c in range(F_IN // FC)]

    gin = params['gin']
    for l in range(2):
        lp = gin[l]
        a = agg2(*h, src, dst, zeros) if l == 0 else agg4(*h, src, dst, zeros)
        layer = _make_tc_layer(len(h), relu_out=True)
        h = list(layer(*h, *a, lp['W1'], lp['b1'].reshape(1, -1),
                       lp['W2'], lp['b2'].reshape(1, -1)))

    lp = gin[2]
    a = agg4(*h, src, dst, zeros)
    final = _make_tc_layer_final(len(h))
    node_out, sums, maxes, counts = final(
        *h, *a, lp['W1'], lp['b1'].reshape(1, -1),
        lp['W2'], lp['b2'].reshape(1, -1),
        params['nc_W'], params['nc_b'].reshape(1, -1), batch_r, batch_c)

    prop_out = _head_call(sums, maxes, counts,
                          params['p1_W'], params['p1_b'].reshape(1, -1),
                          params['p2_W'], params['p2_b'].reshape(1, -1))
    return node_out, prop_out


# speedup vs baseline: 1.3625x; 1.3625x over previous
import functools

import jax
import jax.numpy as jnp
from jax import lax
from jax.experimental import pallas as pl
from jax.experimental.pallas import tpu as pltpu
from jax.experimental.pallas import tpu_sc as plsc

N = 10000
E = 160000
F_IN = 256
H = 512
C_OUT = 64
OUT = 32
G = 64

NUM_CORES = 2
NUM_SUBCORES = 16

FC = 128
K_EDGE = 80
EPT = 10080
E_PAD = NUM_SUBCORES * EPT
NB = EPT // K_EDGE
GB = 18
NBG = NB // GB
ACC_ROWS = N + 8
RPT_A = 640
RPT_LAST = N - (NUM_SUBCORES - 1) * RPT_A

BR = 1000


def _make_sc_agg(num_chunks):
    ppc = num_chunks // NUM_CORES
    mesh = plsc.VectorSubcoreMesh(core_axis_name="c", subcore_axis_name="s")

    @functools.partial(
        pl.kernel,
        mesh=mesh,
        out_type=[jax.ShapeDtypeStruct((N, FC), jnp.float32)
                  for _ in range(num_chunks)],
        scratch_types=[
            pltpu.VMEM((GB, K_EDGE), jnp.int32),
            pltpu.VMEM((GB, K_EDGE), jnp.int32),
            pltpu.VMEM((K_EDGE, FC), jnp.float32),
            pltpu.VMEM((K_EDGE, FC), jnp.float32),
            pltpu.VMEM_SHARED((ACC_ROWS, FC), jnp.float32),
            pltpu.SemaphoreType.DMA,
            pltpu.SemaphoreType.DMA,
            pltpu.SemaphoreType.DMA,
            pltpu.SemaphoreType.DMA,
        ],
    )
    def agg_kernel(*refs):
        hs = refs[:num_chunks]
        src_hbm = refs[num_chunks]
        dst_hbm = refs[num_chunks + 1]
        zeros_hbm = refs[num_chunks + 2]
        outs = refs[num_chunks + 3:2 * num_chunks + 3]
        (src_g, dst_g, rows0, rows1, acc,
         sem0, sem1, ssem0, ssem1) = refs[2 * num_chunks + 3:]

        core = lax.axis_index("c")
        sub = lax.axis_index("s")
        r0 = sub * RPT_A

        def rows_copy(src_ref, dst_ref):
            @pl.when(sub < NUM_SUBCORES - 1)
            def _():
                pltpu.sync_copy(src_ref.at[pl.ds(r0, RPT_A)],
                                dst_ref.at[pl.ds(r0, RPT_A)])

            @pl.when(sub == NUM_SUBCORES - 1)
            def _():
                pltpu.sync_copy(src_ref.at[pl.ds(r0, RPT_LAST)],
                                dst_ref.at[pl.ds(r0, RPT_LAST)])

        def edge_loop(h_hbm):
            def g_copy(j, rows, sem):
                return pltpu.make_async_copy(h_hbm.at[src_g.at[j]],
                                             rows, sem)

            class _Scatter:
                def __init__(self, j, rows, sem):
                    self.d = pltpu.make_async_copy(rows, acc.at[dst_g.at[j]],
                                                   sem)

                def start(self):
                    self.d.start(add=True)

                def wait(self):
                    self.d.wait()

            def s_copy(j, rows, sem):
                return _Scatter(j, rows, sem)

            assert GB % 2 == 0
            for g in range(NBG):
                pltpu.sync_copy(src_hbm.at[sub, g], src_g)
                pltpu.sync_copy(dst_hbm.at[sub, g], dst_g)
                g_copy(0, rows0, sem0).start()
                g_copy(0, rows0, sem0).wait()
                s_copy(0, rows0, ssem0).start()
                g_copy(1, rows1, sem1).start()

                def pair(jo, carry):
                    j1 = 2 * jo + 1
                    g_copy(j1, rows1, sem1).wait()
                    s_copy(j1 - 1, rows0, ssem0).wait()
                    s_copy(j1, rows1, ssem1).start()
                    g_copy(j1 + 1, rows0, sem0).start()
                    g_copy(j1 + 1, rows0, sem0).wait()
                    s_copy(j1, rows1, ssem1).wait()
                    s_copy(j1 + 1, rows0, ssem0).start()
                    g_copy(j1 + 2, rows1, sem1).start()
                    return carry

                lax.fori_loop(0, GB // 2 - 1, pair, 0)
                g_copy(GB - 1, rows1, sem1).wait()
                s_copy(GB - 2, rows0, ssem0).wait()
                s_copy(GB - 1, rows1, ssem1).start()
                s_copy(GB - 1, rows1, ssem1).wait()

        for p in range(ppc):
            rows_copy(zeros_hbm, acc)
            plsc.subcore_barrier()

            @pl.when(core == 0)
            def _():
                edge_loop(hs[p])

            @pl.when(core == 1)
            def _():
                edge_loop(hs[ppc + p])

            plsc.subcore_barrier()

            @pl.when(core == 0)
            def _():
                rows_copy(acc, outs[p])

            @pl.when(core == 1)
            def _():
                rows_copy(acc, outs[ppc + p])

            plsc.subcore_barrier()

    return agg_kernel


def _make_tc_layer(ch_in, relu_out):
    fi = ch_in * FC
    n_out = H // FC

    def body(*refs):
        hs = refs[:ch_in]
        ags = refs[ch_in:2 * ch_in]
        w1, b1, w2, b2 = refs[2 * ch_in:2 * ch_in + 4]
        outs = refs[2 * ch_in + 4:]
        acc = None
        for c in range(ch_in):
            xin = hs[c][...] + ags[c][...]
            part = jnp.dot(xin, w1[c * FC:(c + 1) * FC, :],
                           preferred_element_type=jnp.float32)
            acc = part if acc is None else acc + part
        m = jnp.maximum(acc + b1[...], 0.0)
        o = jnp.dot(m, w2[...], preferred_element_type=jnp.float32) + b2[...]
        if relu_out:
            o = jnp.maximum(o, 0.0)
        for c in range(n_out):
            outs[c][...] = o[:, c * FC:(c + 1) * FC]

    row_spec = pl.BlockSpec((BR, FC), lambda i: (i, 0))
    in_specs = ([row_spec] * (2 * ch_in) + [
        pl.BlockSpec((fi, H), lambda i: (0, 0)),
        pl.BlockSpec((1, H), lambda i: (0, 0)),
        pl.BlockSpec((H, H), lambda i: (0, 0)),
        pl.BlockSpec((1, H), lambda i: (0, 0)),
    ])
    return pl.pallas_call(
        body,
        grid=(N // BR,),
        in_specs=in_specs,
        out_specs=[row_spec] * n_out,
        out_shape=[jax.ShapeDtypeStruct((N, FC), jnp.float32)
                   for _ in range(n_out)],
    )


def _make_tc_layer_final(ch_in):
    fi = ch_in * FC

    def body(*refs):
        hs = refs[:ch_in]
        ags = refs[ch_in:2 * ch_in]
        (w1, b1, w2, b2, ncw, ncb, batch_r, batch_c,
         node_ref, sums_ref, maxes_ref, counts_ref) = refs[2 * ch_in:]

        acc = None
        for c in range(ch_in):
            xin = hs[c][...] + ags[c][...]
            part = jnp.dot(xin, w1[c * FC:(c + 1) * FC, :],
                           preferred_element_type=jnp.float32)
            acc = part if acc is None else acc + part
        m = jnp.maximum(acc + b1[...], 0.0)
        o = jnp.dot(m, w2[...], preferred_element_type=jnp.float32) + b2[...]

        node_ref[...] = (jnp.dot(jnp.maximum(o, 0.0), ncw[...],
                                 preferred_element_type=jnp.float32)
                         + ncb[...])

        i = pl.program_id(0)

        @pl.when(i == 0)
        def _():
            sums_ref[...] = jnp.zeros_like(sums_ref)
            counts_ref[...] = jnp.zeros_like(counts_ref)
            maxes_ref[...] = jnp.full_like(maxes_ref, -1e30)

        ids_row = batch_r[0]
        ids_col = batch_c[0]
        onehot = (lax.broadcasted_iota(jnp.int32, (G, BR), 0)
                  == ids_row).astype(jnp.float32)
        sums_ref[...] += jnp.dot(onehot, o,
                                 preferred_element_type=jnp.float32)
        cnt = jnp.sum(onehot, axis=1, keepdims=True)
        counts_ref[...] += jnp.broadcast_to(cnt, counts_ref.shape)

        def mbody(g, cur):
            msk = ids_col == g
            mx = jnp.max(jnp.where(msk, o, -1e30), axis=0, keepdims=True)
            sel = lax.broadcasted_iota(jnp.int32, (G, 1), 0) == g
            return jnp.where(sel, jnp.maximum(cur, mx), cur)

        maxes_ref[...] = lax.fori_loop(ids_col[0, 0], ids_col[BR - 1, 0] + 1,
                                       mbody, maxes_ref[...])

    row_spec = pl.BlockSpec((BR, FC), lambda i: (i, 0))
    full = lambda shape: pl.BlockSpec(shape, lambda i: tuple(0 for _ in shape))
    in_specs = ([row_spec] * (2 * ch_in) + [
        full((fi, H)),
        full((1, H)),
        full((H, H)),
        full((1, H)),
        full((H, C_OUT)),
        full((1, C_OUT)),
        pl.BlockSpec((1, 1, BR), lambda i: (i, 0, 0)),
        pl.BlockSpec((1, BR, 1), lambda i: (i, 0, 0)),
    ])
    out_specs = [
        pl.BlockSpec((BR, C_OUT), lambda i: (i, 0)),
        full((G, H)),
        full((G, H)),
        full((G, FC)),
    ]
    out_shape = [
        jax.ShapeDtypeStruct((N, C_OUT), jnp.float32),
        jax.ShapeDtypeStruct((G, H), jnp.float32),
        jax.ShapeDtypeStruct((G, H), jnp.float32),
        jax.ShapeDtypeStruct((G, FC), jnp.float32),
    ]
    return pl.pallas_call(
        body,
        grid=(N // BR,),
        in_specs=in_specs,
        out_specs=out_specs,
        out_shape=out_shape,
    )


def _head_body(sums_ref, maxes_ref, counts_ref, p1w, p1b, p2w, p2b, out_ref):
    cnt = counts_ref[...][:, 0:1]
    mean = sums_ref[...] / jnp.maximum(cnt, 1.0)
    mx = jnp.where(cnt > 0.0, maxes_ref[...], 0.0)
    gcat = jnp.concatenate([mean, mx], axis=1)
    p = jnp.maximum(jnp.dot(gcat, p1w[...],
                            preferred_element_type=jnp.float32) + p1b[...],
                    0.0)
    out_ref[...] = (jnp.dot(p, p2w[...],
                            preferred_element_type=jnp.float32) + p2b[...])


_head_call = pl.pallas_call(
    _head_body,
    out_shape=jax.ShapeDtypeStruct((G, OUT), jnp.float32),
)


def kernel(x, edge_index, batch, params):
    pad = E_PAD - E
    src = jnp.concatenate(
        [edge_index[0], jnp.zeros((pad,), jnp.int32)]
    ).reshape(NUM_SUBCORES, NBG, GB, K_EDGE)
    dst = jnp.concatenate(
        [edge_index[1], jnp.full((pad,), N, jnp.int32)]
    ).reshape(NUM_SUBCORES, NBG, GB, K_EDGE)
    zeros = jnp.zeros((N, FC), jnp.float32)
    batch_r = batch.reshape(N // BR, 1, BR)
    batch_c = batch.reshape(N // BR, BR, 1)

    agg2 = _make_sc_agg(2)
    agg4 = _make_sc_agg(4)

    h = [x[:, c * FC:(c + 1) * FC] for c in range(F_IN // FC)]

    gin = params['gin']
    for l in range(2):
        lp = gin[l]
        a = agg2(*h, src, dst, zeros) if l == 0 else agg4(*h, src, dst, zeros)
        layer = _make_tc_layer(len(h), relu_out=True)
        h = list(layer(*h, *a, lp['W1'], lp['b1'].reshape(1, -1),
                       lp['W2'], lp['b2'].reshape(1, -1)))

    lp = gin[2]
    a = agg4(*h, src, dst, zeros)
    final = _make_tc_layer_final(len(h))
    node_out, sums, maxes, counts = final(
        *h, *a, lp['W1'], lp['b1'].reshape(1, -1),
        lp['W2'], lp['b2'].reshape(1, -1),
        params['nc_W'], params['nc_b'].reshape(1, -1), batch_r, batch_c)

    prop_out = _head_call(sums, maxes, counts,
                          params['p1_W'], params['p1_b'].reshape(1, -1),
                          params['p2_W'], params['p2_b'].reshape(1, -1))
    return node_out, prop_out


# speedup vs baseline: 1.7562x; 1.2889x over previous
import functools

import jax
import jax.numpy as jnp
from jax import lax
from jax.experimental import pallas as pl
from jax.experimental.pallas import tpu as pltpu
from jax.experimental.pallas import tpu_sc as plsc

N = 10000
E = 160000
F_IN = 256
H = 512
C_OUT = 64
OUT = 32
G = 64

NUM_CORES = 2
NUM_SUBCORES = 16

FC = 128
K_EDGE = 80
EPT = E // NUM_SUBCORES
E_PAD = NUM_SUBCORES * EPT
NB = EPT // K_EDGE
GB = 25
NBG = NB // GB
ACC_ROWS = N + 8
RPT_A = 640
RPT_LAST = N - (NUM_SUBCORES - 1) * RPT_A

BR = 1000


def _make_sc_agg(num_chunks):
    ppc = num_chunks // NUM_CORES
    mesh = plsc.VectorSubcoreMesh(core_axis_name="c", subcore_axis_name="s")

    @functools.partial(
        pl.kernel,
        mesh=mesh,
        out_type=[jax.ShapeDtypeStruct((N, FC), jnp.float32)
                  for _ in range(num_chunks)],
        scratch_types=[
            pltpu.VMEM((GB, K_EDGE), jnp.int32),
            pltpu.VMEM((GB, K_EDGE), jnp.int32),
            pltpu.VMEM((K_EDGE, FC), jnp.float32),
            pltpu.VMEM((K_EDGE, FC), jnp.float32),
            pltpu.VMEM_SHARED((ACC_ROWS, FC), jnp.float32),
            pltpu.SemaphoreType.DMA,
            pltpu.SemaphoreType.DMA,
            pltpu.SemaphoreType.DMA,
            pltpu.SemaphoreType.DMA,
        ],
    )
    def agg_kernel(*refs):
        hs = refs[:num_chunks]
        src_hbm = refs[num_chunks]
        dst_hbm = refs[num_chunks + 1]
        zeros_hbm = refs[num_chunks + 2]
        outs = refs[num_chunks + 3:2 * num_chunks + 3]
        (src_g, dst_g, rows0, rows1, acc,
         sem0, sem1, ssem0, ssem1) = refs[2 * num_chunks + 3:]

        core = lax.axis_index("c")
        sub = lax.axis_index("s")
        r0 = sub * RPT_A

        def rows_copy(src_ref, dst_ref):
            @pl.when(sub < NUM_SUBCORES - 1)
            def _():
                pltpu.sync_copy(src_ref.at[pl.ds(r0, RPT_A)],
                                dst_ref.at[pl.ds(r0, RPT_A)])

            @pl.when(sub == NUM_SUBCORES - 1)
            def _():
                pltpu.sync_copy(src_ref.at[pl.ds(r0, RPT_LAST)],
                                dst_ref.at[pl.ds(r0, RPT_LAST)])

        def edge_loop(h_hbm):
            def g_copy(j, rows, sem):
                return pltpu.make_async_copy(h_hbm.at[src_g.at[j]],
                                             rows, sem)

            class _Scatter:
                def __init__(self, j, rows, sem):
                    self.d = pltpu.make_async_copy(rows, acc.at[dst_g.at[j]],
                                                   sem)

                def start(self):
                    self.d.start(add=True)

                def wait(self):
                    self.d.wait()

            def s_copy(j, rows, sem):
                return _Scatter(j, rows, sem)

            for g in range(NBG):
                pltpu.sync_copy(src_hbm.at[sub, g], src_g)
                pltpu.sync_copy(dst_hbm.at[sub, g], dst_g)
                g_copy(0, rows0, sem0).start()
                g_copy(0, rows0, sem0).wait()
                s_copy(0, rows0, ssem0).start()
                g_copy(1, rows1, sem1).start()

                def pair(jo, carry):
                    j1 = 2 * jo + 1
                    g_copy(j1, rows1, sem1).wait()
                    s_copy(j1 - 1, rows0, ssem0).wait()
                    s_copy(j1, rows1, ssem1).start()
                    g_copy(j1 + 1, rows0, sem0).start()
                    g_copy(j1 + 1, rows0, sem0).wait()
                    s_copy(j1, rows1, ssem1).wait()
                    s_copy(j1 + 1, rows0, ssem0).start()
                    g_copy(j1 + 2, rows1, sem1).start()
                    return carry

                if GB % 2 == 0:
                    lax.fori_loop(0, GB // 2 - 1, pair, 0)
                    g_copy(GB - 1, rows1, sem1).wait()
                    s_copy(GB - 2, rows0, ssem0).wait()
                    s_copy(GB - 1, rows1, ssem1).start()
                    s_copy(GB - 1, rows1, ssem1).wait()
                else:
                    lax.fori_loop(0, (GB - 3) // 2, pair, 0)
                    g_copy(GB - 2, rows1, sem1).wait()
                    s_copy(GB - 3, rows0, ssem0).wait()
                    s_copy(GB - 2, rows1, ssem1).start()
                    g_copy(GB - 1, rows0, sem0).start()
                    g_copy(GB - 1, rows0, sem0).wait()
                    s_copy(GB - 2, rows1, ssem1).wait()
                    s_copy(GB - 1, rows0, ssem0).start()
                    s_copy(GB - 1, rows0, ssem0).wait()

        for p in range(ppc):
            rows_copy(zeros_hbm, acc)
            plsc.subcore_barrier()

            @pl.when(core == 0)
            def _():
                edge_loop(hs[p])

            @pl.when(core == 1)
            def _():
                edge_loop(hs[ppc + p])

            plsc.subcore_barrier()

            @pl.when(core == 0)
            def _():
                rows_copy(acc, outs[p])

            @pl.when(core == 1)
            def _():
                rows_copy(acc, outs[ppc + p])

            plsc.subcore_barrier()

    return agg_kernel


def _make_tc_layer(ch_in, relu_out):
    fi = ch_in * FC
    n_out = H // FC

    def body(*refs):
        hs = refs[:ch_in]
        ags = refs[ch_in:2 * ch_in]
        w1, b1, w2, b2 = refs[2 * ch_in:2 * ch_in + 4]
        outs = refs[2 * ch_in + 4:]
        acc = None
        for c in range(ch_in):
            xin = hs[c][...] + ags[c][...]
            part = jnp.dot(xin, w1[c * FC:(c + 1) * FC, :],
                           preferred_element_type=jnp.float32)
            acc = part if acc is None else acc + part
        m = jnp.maximum(acc + b1[...], 0.0)
        o = jnp.dot(m, w2[...], preferred_element_type=jnp.float32) + b2[...]
        if relu_out:
            o = jnp.maximum(o, 0.0)
        for c in range(n_out):
            outs[c][...] = o[:, c * FC:(c + 1) * FC]

    row_spec = pl.BlockSpec((BR, FC), lambda i: (i, 0))
    in_specs = ([row_spec] * (2 * ch_in) + [
        pl.BlockSpec((fi, H), lambda i: (0, 0)),
        pl.BlockSpec((1, H), lambda i: (0, 0)),
        pl.BlockSpec((H, H), lambda i: (0, 0)),
        pl.BlockSpec((1, H), lambda i: (0, 0)),
    ])
    return pl.pallas_call(
        body,
        grid=(N // BR,),
        in_specs=in_specs,
        out_specs=[row_spec] * n_out,
        out_shape=[jax.ShapeDtypeStruct((N, FC), jnp.float32)
                   for _ in range(n_out)],
    )


def _make_tc_layer_final(ch_in):
    fi = ch_in * FC

    def body(*refs):
        hs = refs[:ch_in]
        ags = refs[ch_in:2 * ch_in]
        (w1, b1, w2, b2, ncw, ncb, batch_r, batch_c,
         node_ref, sums_ref, maxes_ref, counts_ref) = refs[2 * ch_in:]

        acc = None
        for c in range(ch_in):
            xin = hs[c][...] + ags[c][...]
            part = jnp.dot(xin, w1[c * FC:(c + 1) * FC, :],
                           preferred_element_type=jnp.float32)
            acc = part if acc is None else acc + part
        m = jnp.maximum(acc + b1[...], 0.0)
        o = jnp.dot(m, w2[...], preferred_element_type=jnp.float32) + b2[...]

        node_ref[...] = (jnp.dot(jnp.maximum(o, 0.0), ncw[...],
                                 preferred_element_type=jnp.float32)
                         + ncb[...])

        i = pl.program_id(0)

        @pl.when(i == 0)
        def _():
            sums_ref[...] = jnp.zeros_like(sums_ref)
            counts_ref[...] = jnp.zeros_like(counts_ref)
            maxes_ref[...] = jnp.full_like(maxes_ref, -1e30)

        ids_row = batch_r[0]
        ids_col = batch_c[0]
        onehot = (lax.broadcasted_iota(jnp.int32, (G, BR), 0)
                  == ids_row).astype(jnp.float32)
        sums_ref[...] += jnp.dot(onehot, o,
                                 preferred_element_type=jnp.float32)
        cnt = jnp.sum(onehot, axis=1, keepdims=True)
        counts_ref[...] += jnp.broadcast_to(cnt, counts_ref.shape)

        def mbody(g, cur):
            msk = ids_col == g
            mx = jnp.max(jnp.where(msk, o, -1e30), axis=0, keepdims=True)
            sel = lax.broadcasted_iota(jnp.int32, (G, 1), 0) == g
            return jnp.where(sel, jnp.maximum(cur, mx), cur)

        maxes_ref[...] = lax.fori_loop(ids_col[0, 0], ids_col[BR - 1, 0] + 1,
                                       mbody, maxes_ref[...])

    row_spec = pl.BlockSpec((BR, FC), lambda i: (i, 0))
    full = lambda shape: pl.BlockSpec(shape, lambda i: tuple(0 for _ in shape))
    in_specs = ([row_spec] * (2 * ch_in) + [
        full((fi, H)),
        full((1, H)),
        full((H, H)),
        full((1, H)),
        full((H, C_OUT)),
        full((1, C_OUT)),
        pl.BlockSpec((1, 1, BR), lambda i: (i, 0, 0)),
        pl.BlockSpec((1, BR, 1), lambda i: (i, 0, 0)),
    ])
    out_specs = [
        pl.BlockSpec((BR, C_OUT), lambda i: (i, 0)),
        full((G, H)),
        full((G, H)),
        full((G, FC)),
    ]
    out_shape = [
        jax.ShapeDtypeStruct((N, C_OUT), jnp.float32),
        jax.ShapeDtypeStruct((G, H), jnp.float32),
        jax.ShapeDtypeStruct((G, H), jnp.float32),
        jax.ShapeDtypeStruct((G, FC), jnp.float32),
    ]
    return pl.pallas_call(
        body,
        grid=(N // BR,),
        in_specs=in_specs,
        out_specs=out_specs,
        out_shape=out_shape,
    )


def _head_body(sums_ref, maxes_ref, counts_ref, p1w, p1b, p2w, p2b, out_ref):
    cnt = counts_ref[...][:, 0:1]
    mean = sums_ref[...] / jnp.maximum(cnt, 1.0)
    mx = jnp.where(cnt > 0.0, maxes_ref[...], 0.0)
    gcat = jnp.concatenate([mean, mx], axis=1)
    p = jnp.maximum(jnp.dot(gcat, p1w[...],
                            preferred_element_type=jnp.float32) + p1b[...],
                    0.0)
    out_ref[...] = (jnp.dot(p, p2w[...],
                            preferred_element_type=jnp.float32) + p2b[...])


_head_call = pl.pallas_call(
    _head_body,
    out_shape=jax.ShapeDtypeStruct((G, OUT), jnp.float32),
)


def kernel(x, edge_index, batch, params):
    pad = E_PAD - E
    src_flat, dst_flat = edge_index[0], edge_index[1]
    if pad:
        pad_dst = N + (jnp.arange(pad, dtype=jnp.int32) % 8)
        src_flat = jnp.concatenate([src_flat,
                                    jnp.zeros((pad,), jnp.int32)])
        dst_flat = jnp.concatenate([dst_flat, pad_dst])
    src = src_flat.reshape(NUM_SUBCORES, NBG, GB, K_EDGE)
    dst = dst_flat.reshape(NUM_SUBCORES, NBG, GB, K_EDGE)
    zeros = jnp.zeros((N, FC), jnp.float32)
    batch_r = batch.reshape(N // BR, 1, BR)
    batch_c = batch.reshape(N // BR, BR, 1)

    agg2 = _make_sc_agg(2)
    agg4 = _make_sc_agg(4)

    h = [x[:, c * FC:(c + 1) * FC] for c in range(F_IN // FC)]

    gin = params['gin']
    for l in range(2):
        lp = gin[l]
        a = agg2(*h, src, dst, zeros) if l == 0 else agg4(*h, src, dst, zeros)
        layer = _make_tc_layer(len(h), relu_out=True)
        h = list(layer(*h, *a, lp['W1'], lp['b1'].reshape(1, -1),
                       lp['W2'], lp['b2'].reshape(1, -1)))

    lp = gin[2]
    a = agg4(*h, src, dst, zeros)
    final = _make_tc_layer_final(len(h))
    node_out, sums, maxes, counts = final(
        *h, *a, lp['W1'], lp['b1'].reshape(1, -1),
        lp['W2'], lp['b2'].reshape(1, -1),
        params['nc_W'], params['nc_b'].reshape(1, -1), batch_r, batch_c)

    prop_out = _head_call(sums, maxes, counts,
                          params['p1_W'], params['p1_b'].reshape(1, -1),
                          params['p2_W'], params['p2_b'].reshape(1, -1))
    return node_out, prop_out


# speedup vs baseline: 2.1518x; 1.2253x over previous
import functools

import jax
import jax.numpy as jnp
from jax import lax
from jax.experimental import pallas as pl
from jax.experimental.pallas import tpu as pltpu
from jax.experimental.pallas import tpu_sc as plsc

N = 10000
E = 160000
F_IN = 256
H = 512
C_OUT = 64
OUT = 32
G = 64

NUM_CORES = 2
NUM_SUBCORES = 16

FC = 128
K_EDGE = 80
EPT = E // NUM_SUBCORES
E_PAD = NUM_SUBCORES * EPT
NB = EPT // K_EDGE
GB = 25
NBG = NB // GB
ACC_ROWS = N + 8
RPT_A = 640
RPT_LAST = N - (NUM_SUBCORES - 1) * RPT_A

BR = 1000


def _make_sc_agg(num_chunks):
    ppc = num_chunks // NUM_CORES
    mesh = plsc.VectorSubcoreMesh(core_axis_name="c", subcore_axis_name="s")

    @functools.partial(
        pl.kernel,
        mesh=mesh,
        out_type=[jax.ShapeDtypeStruct((N, FC), jnp.float32)
                  for _ in range(num_chunks)],
        scratch_types=[
            pltpu.VMEM((GB, K_EDGE), jnp.int32),
            pltpu.VMEM((GB, K_EDGE), jnp.int32),
            pltpu.VMEM((K_EDGE, FC), jnp.float32),
            pltpu.VMEM((K_EDGE, FC), jnp.float32),
            pltpu.VMEM_SHARED((ACC_ROWS, FC), jnp.float32),
            pltpu.SemaphoreType.DMA,
            pltpu.SemaphoreType.DMA,
        ],
    )
    def agg_kernel(*refs):
        hs = refs[:num_chunks]
        src_hbm = refs[num_chunks]
        dst_hbm = refs[num_chunks + 1]
        zeros_hbm = refs[num_chunks + 2]
        outs = refs[num_chunks + 3:2 * num_chunks + 3]
        (src_g, dst_g, rows0, rows1, acc,
         sem0, sem1) = refs[2 * num_chunks + 3:]

        core = lax.axis_index("c")
        sub = lax.axis_index("s")
        r0 = sub * RPT_A

        def rows_copy(src_ref, dst_ref):
            @pl.when(sub < NUM_SUBCORES - 1)
            def _():
                pltpu.sync_copy(src_ref.at[pl.ds(r0, RPT_A)],
                                dst_ref.at[pl.ds(r0, RPT_A)])

            @pl.when(sub == NUM_SUBCORES - 1)
            def _():
                pltpu.sync_copy(src_ref.at[pl.ds(r0, RPT_LAST)],
                                dst_ref.at[pl.ds(r0, RPT_LAST)])

        def edge_loop(h_hbm):
            def g_copy(j, rows, sem):
                return pltpu.make_async_copy(h_hbm.at[src_g.at[j]],
                                             rows, sem)

            def scatter(j, rows):
                pltpu.sync_copy(rows, acc.at[dst_g.at[j]], add=True)

            for g in range(NBG):
                pltpu.sync_copy(src_hbm.at[sub, g], src_g)
                pltpu.sync_copy(dst_hbm.at[sub, g], dst_g)
                g_copy(0, rows0, sem0).start()

                def pair(jo, carry):
                    j0 = 2 * jo
                    g_copy(j0 + 1, rows1, sem1).start()
                    g_copy(j0, rows0, sem0).wait()
                    scatter(j0, rows0)
                    g_copy(j0 + 2, rows0, sem0).start()
                    g_copy(j0 + 1, rows1, sem1).wait()
                    scatter(j0 + 1, rows1)
                    return carry

                if GB % 2:
                    lax.fori_loop(0, (GB - 1) // 2, pair, 0)
                    g_copy(GB - 1, rows0, sem0).wait()
                    scatter(GB - 1, rows0)
                else:
                    lax.fori_loop(0, GB // 2 - 1, pair, 0)
                    g_copy(GB - 1, rows1, sem1).start()
                    g_copy(GB - 2, rows0, sem0).wait()
                    scatter(GB - 2, rows0)
                    g_copy(GB - 1, rows1, sem1).wait()
                    scatter(GB - 1, rows1)

        for p in range(ppc):
            rows_copy(zeros_hbm, acc)
            plsc.subcore_barrier()

            @pl.when(core == 0)
            def _():
                edge_loop(hs[p])

            @pl.when(core == 1)
            def _():
                edge_loop(hs[ppc + p])

            plsc.subcore_barrier()

            @pl.when(core == 0)
            def _():
                rows_copy(acc, outs[p])

            @pl.when(core == 1)
            def _():
                rows_copy(acc, outs[ppc + p])

            plsc.subcore_barrier()

    return agg_kernel


def _make_tc_layer(ch_in, relu_out):
    fi = ch_in * FC
    n_out = H // FC

    def body(*refs):
        hs = refs[:ch_in]
        ags = refs[ch_in:2 * ch_in]
        w1, b1, w2, b2 = refs[2 * ch_in:2 * ch_in + 4]
        outs = refs[2 * ch_in + 4:]
        acc = None
        for c in range(ch_in):
            xin = hs[c][...] + ags[c][...]
            part = jnp.dot(xin, w1[c * FC:(c + 1) * FC, :],
                           preferred_element_type=jnp.float32)
            acc = part if acc is None else acc + part
        m = jnp.maximum(acc + b1[...], 0.0)
        o = jnp.dot(m, w2[...], preferred_element_type=jnp.float32) + b2[...]
        if relu_out:
            o = jnp.maximum(o, 0.0)
        for c in range(n_out):
            outs[c][...] = o[:, c * FC:(c + 1) * FC]

    row_spec = pl.BlockSpec((BR, FC), lambda i: (i, 0))
    in_specs = ([row_spec] * (2 * ch_in) + [
        pl.BlockSpec((fi, H), lambda i: (0, 0)),
        pl.BlockSpec((1, H), lambda i: (0, 0)),
        pl.BlockSpec((H, H), lambda i: (0, 0)),
        pl.BlockSpec((1, H), lambda i: (0, 0)),
    ])
    return pl.pallas_call(
        body,
        grid=(N // BR,),
        in_specs=in_specs,
        out_specs=[row_spec] * n_out,
        out_shape=[jax.ShapeDtypeStruct((N, FC), jnp.float32)
                   for _ in range(n_out)],
    )


def _make_tc_layer_final(ch_in):
    fi = ch_in * FC

    def body(*refs):
        hs = refs[:ch_in]
        ags = refs[ch_in:2 * ch_in]
        (w1, b1, w2, b2, ncw, ncb, p1w, p1b, p2w, p2b, batch_r, batch_c,
         node_ref, sums_ref, maxes_ref, counts_ref, prop_ref) = refs[2 * ch_in:]

        acc = None
        for c in range(ch_in):
            xin = hs[c][...] + ags[c][...]
            part = jnp.dot(xin, w1[c * FC:(c + 1) * FC, :],
                           preferred_element_type=jnp.float32)
            acc = part if acc is None else acc + part
        m = jnp.maximum(acc + b1[...], 0.0)
        o = jnp.dot(m, w2[...], preferred_element_type=jnp.float32) + b2[...]

        node_ref[...] = (jnp.dot(jnp.maximum(o, 0.0), ncw[...],
                                 preferred_element_type=jnp.float32)
                         + ncb[...])

        i = pl.program_id(0)

        @pl.when(i == 0)
        def _():
            sums_ref[...] = jnp.zeros_like(sums_ref)
            counts_ref[...] = jnp.zeros_like(counts_ref)
            maxes_ref[...] = jnp.full_like(maxes_ref, -1e30)

        ids_row = batch_r[0]
        ids_col = batch_c[0]
        onehot = (lax.broadcasted_iota(jnp.int32, (G, BR), 0)
                  == ids_row).astype(jnp.float32)
        sums_ref[...] += jnp.dot(onehot, o,
                                 preferred_element_type=jnp.float32)
        cnt = jnp.sum(onehot, axis=1, keepdims=True)
        counts_ref[...] += jnp.broadcast_to(cnt, counts_ref.shape)

        def mbody(g, cur):
            msk = ids_col == g
            mx = jnp.max(jnp.where(msk, o, -1e30), axis=0, keepdims=True)
            sel = lax.broadcasted_iota(jnp.int32, (G, 1), 0) == g
            return jnp.where(sel, jnp.maximum(cur, mx), cur)

        maxes = lax.fori_loop(ids_col[0, 0], ids_col[BR - 1, 0] + 1,
                              mbody, maxes_ref[...])
        maxes_ref[...] = maxes

        @pl.when(i == N // BR - 1)
        def _():
            cnt = counts_ref[...][:, 0:1]
            mean = sums_ref[...] / jnp.maximum(cnt, 1.0)
            mx = jnp.where(cnt > 0.0, maxes, 0.0)
            gcat = jnp.concatenate([mean, mx], axis=1)
            p = jnp.maximum(
                jnp.dot(gcat, p1w[...],
                        preferred_element_type=jnp.float32) + p1b[...], 0.0)
            prop_ref[...] = (jnp.dot(p, p2w[...],
                                     preferred_element_type=jnp.float32)
                             + p2b[...])

    row_spec = pl.BlockSpec((BR, FC), lambda i: (i, 0))
    full = lambda shape: pl.BlockSpec(shape, lambda i: tuple(0 for _ in shape))
    in_specs = ([row_spec] * (2 * ch_in) + [
        full((fi, H)),
        full((1, H)),
        full((H, H)),
        full((1, H)),
        full((H, C_OUT)),
        full((1, C_OUT)),
        full((2 * H, H)),
        full((1, H)),
        full((H, OUT)),
        full((1, OUT)),
        pl.BlockSpec((1, 1, BR), lambda i: (i, 0, 0)),
        pl.BlockSpec((1, BR, 1), lambda i: (i, 0, 0)),
    ])
    out_specs = [
        pl.BlockSpec((BR, C_OUT), lambda i: (i, 0)),
        full((G, H)),
        full((G, H)),
        full((G, FC)),
        full((G, OUT)),
    ]
    out_shape = [
        jax.ShapeDtypeStruct((N, C_OUT), jnp.float32),
        jax.ShapeDtypeStruct((G, H), jnp.float32),
        jax.ShapeDtypeStruct((G, H), jnp.float32),
        jax.ShapeDtypeStruct((G, FC), jnp.float32),
        jax.ShapeDtypeStruct((G, OUT), jnp.float32),
    ]
    return pl.pallas_call(
        body,
        grid=(N // BR,),
        in_specs=in_specs,
        out_specs=out_specs,
        out_shape=out_shape,
    )


def kernel(x, edge_index, batch, params):
    pad = E_PAD - E
    src_flat, dst_flat = edge_index[0], edge_index[1]
    if pad:
        pad_dst = N + (jnp.arange(pad, dtype=jnp.int32) % 8)
        src_flat = jnp.concatenate([src_flat,
                                    jnp.zeros((pad,), jnp.int32)])
        dst_flat = jnp.concatenate([dst_flat, pad_dst])
    src = src_flat.reshape(NUM_SUBCORES, NBG, GB, K_EDGE)
    dst = dst_flat.reshape(NUM_SUBCORES, NBG, GB, K_EDGE)
    zeros = jnp.zeros((N, FC), jnp.float32)
    batch_r = batch.reshape(N // BR, 1, BR)
    batch_c = batch.reshape(N // BR, BR, 1)

    agg2 = _make_sc_agg(2)
    agg4 = _make_sc_agg(4)

    h = [x[:, c * FC:(c + 1) * FC] for c in range(F_IN // FC)]

    gin = params['gin']
    for l in range(2):
        lp = gin[l]
        a = agg2(*h, src, dst, zeros) if l == 0 else agg4(*h, src, dst, zeros)
        layer = _make_tc_layer(len(h), relu_out=True)
        h = list(layer(*h, *a, lp['W1'], lp['b1'].reshape(1, -1),
                       lp['W2'], lp['b2'].reshape(1, -1)))

    lp = gin[2]
    a = agg4(*h, src, dst, zeros)
    final = _make_tc_layer_final(len(h))
    node_out, _sums, _maxes, _counts, prop_out = final(
        *h, *a, lp['W1'], lp['b1'].reshape(1, -1),
        lp['W2'], lp['b2'].reshape(1, -1),
        params['nc_W'], params['nc_b'].reshape(1, -1),
        params['p1_W'], params['p1_b'].reshape(1, -1),
        params['p2_W'], params['p2_b'].reshape(1, -1),
        batch_r, batch_c)
    return node_out, prop_out


# speedup vs baseline: 2.2221x; 1.0327x over previous
import functools

import jax
import jax.numpy as jnp
from jax import lax
from jax.experimental import pallas as pl
from jax.experimental.pallas import tpu as pltpu
from jax.experimental.pallas import tpu_sc as plsc

N = 10000
E = 160000
F_IN = 256
H = 512
C_OUT = 64
OUT = 32
G = 64

NUM_CORES = 2
NUM_SUBCORES = 16

FC = 128
K_EDGE = 80
EPT = E // NUM_SUBCORES
E_PAD = NUM_SUBCORES * EPT
NB = EPT // K_EDGE
GB = 25
NBG = NB // GB
ACC_ROWS = N + 8
RPT_A = 640
RPT_LAST = N - (NUM_SUBCORES - 1) * RPT_A

BR = 1000


def _make_sc_agg(num_chunks):
    ppc = num_chunks // NUM_CORES
    mesh = plsc.VectorSubcoreMesh(core_axis_name="c", subcore_axis_name="s")

    @functools.partial(
        pl.kernel,
        mesh=mesh,
        out_type=[jax.ShapeDtypeStruct((N, FC), jnp.float32)
                  for _ in range(num_chunks)],
        scratch_types=[
            pltpu.VMEM((GB, K_EDGE), jnp.int32),
            pltpu.VMEM((GB, K_EDGE), jnp.int32),
            pltpu.VMEM((GB, K_EDGE), jnp.int32),
            pltpu.VMEM((GB, K_EDGE), jnp.int32),
            pltpu.VMEM((K_EDGE, FC), jnp.float32),
            pltpu.VMEM((K_EDGE, FC), jnp.float32),
            pltpu.VMEM_SHARED((ACC_ROWS, FC), jnp.float32),
            pltpu.SemaphoreType.DMA,
            pltpu.SemaphoreType.DMA,
            pltpu.SemaphoreType.DMA,
            pltpu.SemaphoreType.DMA,
        ],
    )
    def agg_kernel(*refs):
        hs = refs[:num_chunks]
        src_hbm = refs[num_chunks]
        dst_hbm = refs[num_chunks + 1]
        zeros_hbm = refs[num_chunks + 2]
        outs = refs[num_chunks + 3:2 * num_chunks + 3]
        (src_a, dst_a, src_b, dst_b, rows0, rows1, acc,
         sem0, sem1, isem_a, isem_b) = refs[2 * num_chunks + 3:]

        core = lax.axis_index("c")
        sub = lax.axis_index("s")
        r0 = sub * RPT_A

        def rows_copy(src_ref, dst_ref):
            @pl.when(sub < NUM_SUBCORES - 1)
            def _():
                pltpu.sync_copy(src_ref.at[pl.ds(r0, RPT_A)],
                                dst_ref.at[pl.ds(r0, RPT_A)])

            @pl.when(sub == NUM_SUBCORES - 1)
            def _():
                pltpu.sync_copy(src_ref.at[pl.ds(r0, RPT_LAST)],
                                dst_ref.at[pl.ds(r0, RPT_LAST)])

        idx_bufs = [(src_a, dst_a, isem_a), (src_b, dst_b, isem_b)]

        def idx_copies(g, sbuf, dbuf, isem):
            return (pltpu.make_async_copy(src_hbm.at[sub, g], sbuf, isem),
                    pltpu.make_async_copy(dst_hbm.at[sub, g], dbuf, isem))

        def edge_loop(h_hbm):
            def g_copy(j, rows, sem, src_g):
                return pltpu.make_async_copy(h_hbm.at[src_g.at[j]],
                                             rows, sem)

            def scatter(j, rows, dst_g):
                pltpu.sync_copy(rows, acc.at[dst_g.at[j]], add=True)

            for c in idx_copies(0, *idx_bufs[0]):
                c.start()
            for g in range(NBG):
                src_g, dst_g, _ = idx_bufs[g % 2]
                for c in idx_copies(g, *idx_bufs[g % 2]):
                    c.wait()
                if g + 1 < NBG:
                    for c in idx_copies(g + 1, *idx_bufs[(g + 1) % 2]):
                        c.start()
                g_copy(0, rows0, sem0, src_g).start()

                def pair(jo, carry):
                    j0 = 2 * jo
                    g_copy(j0 + 1, rows1, sem1, src_g).start()
                    g_copy(j0, rows0, sem0, src_g).wait()
                    scatter(j0, rows0, dst_g)
                    g_copy(j0 + 2, rows0, sem0, src_g).start()
                    g_copy(j0 + 1, rows1, sem1, src_g).wait()
                    scatter(j0 + 1, rows1, dst_g)
                    return carry

                if GB % 2:
                    lax.fori_loop(0, (GB - 1) // 2, pair, 0)
                    g_copy(GB - 1, rows0, sem0, src_g).wait()
                    scatter(GB - 1, rows0, dst_g)
                else:
                    lax.fori_loop(0, GB // 2 - 1, pair, 0)
                    g_copy(GB - 1, rows1, sem1, src_g).start()
                    g_copy(GB - 2, rows0, sem0, src_g).wait()
                    scatter(GB - 2, rows0, dst_g)
                    g_copy(GB - 1, rows1, sem1, src_g).wait()
                    scatter(GB - 1, rows1, dst_g)

        for p in range(ppc):
            rows_copy(zeros_hbm, acc)
            plsc.subcore_barrier()

            @pl.when(core == 0)
            def _():
                edge_loop(hs[p])

            @pl.when(core == 1)
            def _():
                edge_loop(hs[ppc + p])

            plsc.subcore_barrier()

            @pl.when(core == 0)
            def _():
                rows_copy(acc, outs[p])

            @pl.when(core == 1)
            def _():
                rows_copy(acc, outs[ppc + p])

            plsc.subcore_barrier()

    return agg_kernel


def _make_tc_layer(ch_in, relu_out):
    fi = ch_in * FC
    n_out = H // FC

    def body(*refs):
        hs = refs[:ch_in]
        ags = refs[ch_in:2 * ch_in]
        w1, b1, w2, b2 = refs[2 * ch_in:2 * ch_in + 4]
        outs = refs[2 * ch_in + 4:]
        acc = None
        for c in range(ch_in):
            xin = hs[c][...] + ags[c][...]
            part = jnp.dot(xin, w1[c * FC:(c + 1) * FC, :],
                           preferred_element_type=jnp.float32)
            acc = part if acc is None else acc + part
        m = jnp.maximum(acc + b1[...], 0.0)
        o = jnp.dot(m, w2[...], preferred_element_type=jnp.float32) + b2[...]
        if relu_out:
            o = jnp.maximum(o, 0.0)
        for c in range(n_out):
            outs[c][...] = o[:, c * FC:(c + 1) * FC]

    row_spec = pl.BlockSpec((BR, FC), lambda i: (i, 0))
    in_specs = ([row_spec] * (2 * ch_in) + [
        pl.BlockSpec((fi, H), lambda i: (0, 0)),
        pl.BlockSpec((1, H), lambda i: (0, 0)),
        pl.BlockSpec((H, H), lambda i: (0, 0)),
        pl.BlockSpec((1, H), lambda i: (0, 0)),
    ])
    return pl.pallas_call(
        body,
        grid=(N // BR,),
        in_specs=in_specs,
        out_specs=[row_spec] * n_out,
        out_shape=[jax.ShapeDtypeStruct((N, FC), jnp.float32)
                   for _ in range(n_out)],
    )


def _make_tc_layer_final(ch_in):
    fi = ch_in * FC

    def body(*refs):
        hs = refs[:ch_in]
        ags = refs[ch_in:2 * ch_in]
        (w1, b1, w2, b2, ncw, ncb, p1w, p1b, p2w, p2b, batch_r, batch_c,
         node_ref, sums_ref, maxes_ref, counts_ref, prop_ref) = refs[2 * ch_in:]

        acc = None
        for c in range(ch_in):
            xin = hs[c][...] + ags[c][...]
            part = jnp.dot(xin, w1[c * FC:(c + 1) * FC, :],
                           preferred_element_type=jnp.float32)
            acc = part if acc is None else acc + part
        m = jnp.maximum(acc + b1[...], 0.0)
        o = jnp.dot(m, w2[...], preferred_element_type=jnp.float32) + b2[...]

        node_ref[...] = (jnp.dot(jnp.maximum(o, 0.0), ncw[...],
                                 preferred_element_type=jnp.float32)
                         + ncb[...])

        i = pl.program_id(0)

        @pl.when(i == 0)
        def _():
            sums_ref[...] = jnp.zeros_like(sums_ref)
            counts_ref[...] = jnp.zeros_like(counts_ref)
            maxes_ref[...] = jnp.full_like(maxes_ref, -1e30)

        ids_row = batch_r[0]
        ids_col = batch_c[0]
        onehot = (lax.broadcasted_iota(jnp.int32, (G, BR), 0)
                  == ids_row).astype(jnp.float32)
        sums_ref[...] += jnp.dot(onehot, o,
                                 preferred_element_type=jnp.float32)
        cnt = jnp.sum(onehot, axis=1, keepdims=True)
        counts_ref[...] += jnp.broadcast_to(cnt, counts_ref.shape)

        def mbody(g, cur):
            msk = ids_col == g
            mx = jnp.max(jnp.where(msk, o, -1e30), axis=0, keepdims=True)
            sel = lax.broadcasted_iota(jnp.int32, (G, 1), 0) == g
            return jnp.where(sel, jnp.maximum(cur, mx), cur)

        maxes = lax.fori_loop(ids_col[0, 0], ids_col[BR - 1, 0] + 1,
                              mbody, maxes_ref[...])
        maxes_ref[...] = maxes

        @pl.when(i == N // BR - 1)
        def _():
            cnt = counts_ref[...][:, 0:1]
            mean = sums_ref[...] / jnp.maximum(cnt, 1.0)
            mx = jnp.where(cnt > 0.0, maxes, 0.0)
            gcat = jnp.concatenate([mean, mx], axis=1)
            p = jnp.maximum(
                jnp.dot(gcat, p1w[...],
                        preferred_element_type=jnp.float32) + p1b[...], 0.0)
            prop_ref[...] = (jnp.dot(p, p2w[...],
                                     preferred_element_type=jnp.float32)
                             + p2b[...])

    row_spec = pl.BlockSpec((BR, FC), lambda i: (i, 0))
    full = lambda shape: pl.BlockSpec(shape, lambda i: tuple(0 for _ in shape))
    in_specs = ([row_spec] * (2 * ch_in) + [
        full((fi, H)),
        full((1, H)),
        full((H, H)),
        full((1, H)),
        full((H, C_OUT)),
        full((1, C_OUT)),
        full((2 * H, H)),
        full((1, H)),
        full((H, OUT)),
        full((1, OUT)),
        pl.BlockSpec((1, 1, BR), lambda i: (i, 0, 0)),
        pl.BlockSpec((1, BR, 1), lambda i: (i, 0, 0)),
    ])
    out_specs = [
        pl.BlockSpec((BR, C_OUT), lambda i: (i, 0)),
        full((G, H)),
        full((G, H)),
        full((G, FC)),
        full((G, OUT)),
    ]
    out_shape = [
        jax.ShapeDtypeStruct((N, C_OUT), jnp.float32),
        jax.ShapeDtypeStruct((G, H), jnp.float32),
        jax.ShapeDtypeStruct((G, H), jnp.float32),
        jax.ShapeDtypeStruct((G, FC), jnp.float32),
        jax.ShapeDtypeStruct((G, OUT), jnp.float32),
    ]
    return pl.pallas_call(
        body,
        grid=(N // BR,),
        in_specs=in_specs,
        out_specs=out_specs,
        out_shape=out_shape,
    )


def kernel(x, edge_index, batch, params):
    pad = E_PAD - E
    src_flat, dst_flat = edge_index[0], edge_index[1]
    if pad:
        pad_dst = N + (jnp.arange(pad, dtype=jnp.int32) % 8)
        src_flat = jnp.concatenate([src_flat,
                                    jnp.zeros((pad,), jnp.int32)])
        dst_flat = jnp.concatenate([dst_flat, pad_dst])
    src = src_flat.reshape(NUM_SUBCORES, NBG, GB, K_EDGE)
    dst = dst_flat.reshape(NUM_SUBCORES, NBG, GB, K_EDGE)
    zeros = jnp.zeros((N, FC), jnp.float32)
    batch_r = batch.reshape(N // BR, 1, BR)
    batch_c = batch.reshape(N // BR, BR, 1)

    agg2 = _make_sc_agg(2)
    agg4 = _make_sc_agg(4)

    h = [x[:, c * FC:(c + 1) * FC] for c in range(F_IN // FC)]

    gin = params['gin']
    for l in range(2):
        lp = gin[l]
        a = agg2(*h, src, dst, zeros) if l == 0 else agg4(*h, src, dst, zeros)
        layer = _make_tc_layer(len(h), relu_out=True)
        h = list(layer(*h, *a, lp['W1'], lp['b1'].reshape(1, -1),
                       lp['W2'], lp['b2'].reshape(1, -1)))

    lp = gin[2]
    a = agg4(*h, src, dst, zeros)
    final = _make_tc_layer_final(len(h))
    node_out, _sums, _maxes, _counts, prop_out = final(
        *h, *a, lp['W1'], lp['b1'].reshape(1, -1),
        lp['W2'], lp['b2'].reshape(1, -1),
        params['nc_W'], params['nc_b'].reshape(1, -1),
        params['p1_W'], params['p1_b'].reshape(1, -1),
        params['p2_W'], params['p2_b'].reshape(1, -1),
        batch_r, batch_c)
    return node_out, prop_out


# speedup vs baseline: 2.4076x; 1.0835x over previous
import functools

import jax
import jax.numpy as jnp
from jax import lax
from jax.experimental import pallas as pl
from jax.experimental.pallas import tpu as pltpu
from jax.experimental.pallas import tpu_sc as plsc

N = 10000
E = 160000
F_IN = 256
H = 512
C_OUT = 64
OUT = 32
G = 64

NUM_CORES = 2
NUM_SUBCORES = 16

FC = 128
K_EDGE = 125
EPT = E // NUM_SUBCORES
E_PAD = NUM_SUBCORES * EPT
NB = EPT // K_EDGE
GB = 20
NBG = NB // GB
ACC_ROWS = N + 8
RPT_A = 640
RPT_LAST = N - (NUM_SUBCORES - 1) * RPT_A

BR = 1000


def _make_sc_agg(num_chunks):
    ppc = num_chunks // NUM_CORES
    mesh = plsc.VectorSubcoreMesh(core_axis_name="c", subcore_axis_name="s")

    @functools.partial(
        pl.kernel,
        mesh=mesh,
        out_type=[jax.ShapeDtypeStruct((N, FC), jnp.float32)
                  for _ in range(num_chunks)],
        scratch_types=[
            pltpu.VMEM((GB, K_EDGE), jnp.int32),
            pltpu.VMEM((GB, K_EDGE), jnp.int32),
            pltpu.VMEM((GB, K_EDGE), jnp.int32),
            pltpu.VMEM((GB, K_EDGE), jnp.int32),
            pltpu.VMEM((K_EDGE, FC), jnp.float32),
            pltpu.VMEM((K_EDGE, FC), jnp.float32),
            pltpu.VMEM_SHARED((ACC_ROWS, FC), jnp.float32),
            pltpu.SemaphoreType.DMA,
            pltpu.SemaphoreType.DMA,
            pltpu.SemaphoreType.DMA,
            pltpu.SemaphoreType.DMA,
        ],
    )
    def agg_kernel(*refs):
        hs = refs[:num_chunks]
        src_hbm = refs[num_chunks]
        dst_hbm = refs[num_chunks + 1]
        zeros_hbm = refs[num_chunks + 2]
        outs = refs[num_chunks + 3:2 * num_chunks + 3]
        (src_a, dst_a, src_b, dst_b, rows0, rows1, acc,
         sem0, sem1, isem_a, isem_b) = refs[2 * num_chunks + 3:]

        core = lax.axis_index("c")
        sub = lax.axis_index("s")
        r0 = sub * RPT_A

        def rows_copy(src_ref, dst_ref):
            @pl.when(sub < NUM_SUBCORES - 1)
            def _():
                pltpu.sync_copy(src_ref.at[pl.ds(r0, RPT_A)],
                                dst_ref.at[pl.ds(r0, RPT_A)])

            @pl.when(sub == NUM_SUBCORES - 1)
            def _():
                pltpu.sync_copy(src_ref.at[pl.ds(r0, RPT_LAST)],
                                dst_ref.at[pl.ds(r0, RPT_LAST)])

        idx_bufs = [(src_a, dst_a, isem_a), (src_b, dst_b, isem_b)]

        def idx_copies(g, sbuf, dbuf, isem):
            return (pltpu.make_async_copy(src_hbm.at[sub, g], sbuf, isem),
                    pltpu.make_async_copy(dst_hbm.at[sub, g], dbuf, isem))

        def edge_loop(h_hbm):
            def g_copy(j, rows, sem, src_g):
                return pltpu.make_async_copy(h_hbm.at[src_g.at[j]],
                                             rows, sem)

            def scatter(j, rows, dst_g):
                pltpu.sync_copy(rows, acc.at[dst_g.at[j]], add=True)

            for c in idx_copies(0, *idx_bufs[0]):
                c.start()
            for g in range(NBG):
                src_g, dst_g, _ = idx_bufs[g % 2]
                for c in idx_copies(g, *idx_bufs[g % 2]):
                    c.wait()
                if g + 1 < NBG:
                    for c in idx_copies(g + 1, *idx_bufs[(g + 1) % 2]):
                        c.start()
                g_copy(0, rows0, sem0, src_g).start()

                def pair(jo, carry):
                    j0 = 2 * jo
                    g_copy(j0 + 1, rows1, sem1, src_g).start()
                    g_copy(j0, rows0, sem0, src_g).wait()
                    scatter(j0, rows0, dst_g)
                    g_copy(j0 + 2, rows0, sem0, src_g).start()
                    g_copy(j0 + 1, rows1, sem1, src_g).wait()
                    scatter(j0 + 1, rows1, dst_g)
                    return carry

                if GB % 2:
                    lax.fori_loop(0, (GB - 1) // 2, pair, 0)
                    g_copy(GB - 1, rows0, sem0, src_g).wait()
                    scatter(GB - 1, rows0, dst_g)
                else:
                    lax.fori_loop(0, GB // 2 - 1, pair, 0)
                    g_copy(GB - 1, rows1, sem1, src_g).start()
                    g_copy(GB - 2, rows0, sem0, src_g).wait()
                    scatter(GB - 2, rows0, dst_g)
                    g_copy(GB - 1, rows1, sem1, src_g).wait()
                    scatter(GB - 1, rows1, dst_g)

        for p in range(ppc):
            rows_copy(zeros_hbm, acc)
            plsc.subcore_barrier()

            @pl.when(core == 0)
            def _():
                edge_loop(hs[p])

            @pl.when(core == 1)
            def _():
                edge_loop(hs[ppc + p])

            plsc.subcore_barrier()

            @pl.when(core == 0)
            def _():
                rows_copy(acc, outs[p])

            @pl.when(core == 1)
            def _():
                rows_copy(acc, outs[ppc + p])

            plsc.subcore_barrier()

    return agg_kernel


def _make_tc_layer(ch_in, relu_out):
    fi = ch_in * FC
    n_out = H // FC

    def body(*refs):
        hs = refs[:ch_in]
        ags = refs[ch_in:2 * ch_in]
        w1, b1, w2, b2 = refs[2 * ch_in:2 * ch_in + 4]
        outs = refs[2 * ch_in + 4:]
        acc = None
        for c in range(ch_in):
            xin = hs[c][...] + ags[c][...]
            part = jnp.dot(xin, w1[c * FC:(c + 1) * FC, :],
                           preferred_element_type=jnp.float32)
            acc = part if acc is None else acc + part
        m = jnp.maximum(acc + b1[...], 0.0)
        o = jnp.dot(m, w2[...], preferred_element_type=jnp.float32) + b2[...]
        if relu_out:
            o = jnp.maximum(o, 0.0)
        for c in range(n_out):
            outs[c][...] = o[:, c * FC:(c + 1) * FC]

    row_spec = pl.BlockSpec((BR, FC), lambda i: (i, 0))
    in_specs = ([row_spec] * (2 * ch_in) + [
        pl.BlockSpec((fi, H), lambda i: (0, 0)),
        pl.BlockSpec((1, H), lambda i: (0, 0)),
        pl.BlockSpec((H, H), lambda i: (0, 0)),
        pl.BlockSpec((1, H), lambda i: (0, 0)),
    ])
    return pl.pallas_call(
        body,
        grid=(N // BR,),
        in_specs=in_specs,
        out_specs=[row_spec] * n_out,
        out_shape=[jax.ShapeDtypeStruct((N, FC), jnp.float32)
                   for _ in range(n_out)],
    )


def _make_tc_layer_final(ch_in):
    fi = ch_in * FC

    def body(*refs):
        hs = refs[:ch_in]
        ags = refs[ch_in:2 * ch_in]
        (w1, b1, w2, b2, ncw, ncb, p1w, p1b, p2w, p2b, batch_r, batch_c,
         node_ref, sums_ref, maxes_ref, counts_ref, prop_ref) = refs[2 * ch_in:]

        acc = None
        for c in range(ch_in):
            xin = hs[c][...] + ags[c][...]
            part = jnp.dot(xin, w1[c * FC:(c + 1) * FC, :],
                           preferred_element_type=jnp.float32)
            acc = part if acc is None else acc + part
        m = jnp.maximum(acc + b1[...], 0.0)
        o = jnp.dot(m, w2[...], preferred_element_type=jnp.float32) + b2[...]

        node_ref[...] = (jnp.dot(jnp.maximum(o, 0.0), ncw[...],
                                 preferred_element_type=jnp.float32)
                         + ncb[...])

        i = pl.program_id(0)

        @pl.when(i == 0)
        def _():
            sums_ref[...] = jnp.zeros_like(sums_ref)
            counts_ref[...] = jnp.zeros_like(counts_ref)
            maxes_ref[...] = jnp.full_like(maxes_ref, -1e30)

        ids_row = batch_r[0]
        ids_col = batch_c[0]
        onehot = (lax.broadcasted_iota(jnp.int32, (G, BR), 0)
                  == ids_row).astype(jnp.float32)
        sums_ref[...] += jnp.dot(onehot, o,
                                 preferred_element_type=jnp.float32)
        cnt = jnp.sum(onehot, axis=1, keepdims=True)
        counts_ref[...] += jnp.broadcast_to(cnt, counts_ref.shape)

        def mbody(g, cur):
            msk = ids_col == g
            mx = jnp.max(jnp.where(msk, o, -1e30), axis=0, keepdims=True)
            sel = lax.broadcasted_iota(jnp.int32, (G, 1), 0) == g
            return jnp.where(sel, jnp.maximum(cur, mx), cur)

        maxes = lax.fori_loop(ids_col[0, 0], ids_col[BR - 1, 0] + 1,
                              mbody, maxes_ref[...])
        maxes_ref[...] = maxes

        @pl.when(i == N // BR - 1)
        def _():
            cnt = counts_ref[...][:, 0:1]
            mean = sums_ref[...] / jnp.maximum(cnt, 1.0)
            mx = jnp.where(cnt > 0.0, maxes, 0.0)
            gcat = jnp.concatenate([mean, mx], axis=1)
            p = jnp.maximum(
                jnp.dot(gcat, p1w[...],
                        preferred_element_type=jnp.float32) + p1b[...], 0.0)
            prop_ref[...] = (jnp.dot(p, p2w[...],
                                     preferred_element_type=jnp.float32)
                             + p2b[...])

    row_spec = pl.BlockSpec((BR, FC), lambda i: (i, 0))
    full = lambda shape: pl.BlockSpec(shape, lambda i: tuple(0 for _ in shape))
    in_specs = ([row_spec] * (2 * ch_in) + [
        full((fi, H)),
        full((1, H)),
        full((H, H)),
        full((1, H)),
        full((H, C_OUT)),
        full((1, C_OUT)),
        full((2 * H, H)),
        full((1, H)),
        full((H, OUT)),
        full((1, OUT)),
        pl.BlockSpec((1, 1, BR), lambda i: (i, 0, 0)),
        pl.BlockSpec((1, BR, 1), lambda i: (i, 0, 0)),
    ])
    out_specs = [
        pl.BlockSpec((BR, C_OUT), lambda i: (i, 0)),
        full((G, H)),
        full((G, H)),
        full((G, FC)),
        full((G, OUT)),
    ]
    out_shape = [
        jax.ShapeDtypeStruct((N, C_OUT), jnp.float32),
        jax.ShapeDtypeStruct((G, H), jnp.float32),
        jax.ShapeDtypeStruct((G, H), jnp.float32),
        jax.ShapeDtypeStruct((G, FC), jnp.float32),
        jax.ShapeDtypeStruct((G, OUT), jnp.float32),
    ]
    return pl.pallas_call(
        body,
        grid=(N // BR,),
        in_specs=in_specs,
        out_specs=out_specs,
        out_shape=out_shape,
    )


def kernel(x, edge_index, batch, params):
    pad = E_PAD - E
    src_flat, dst_flat = edge_index[0], edge_index[1]
    if pad:
        pad_dst = N + (jnp.arange(pad, dtype=jnp.int32) % 8)
        src_flat = jnp.concatenate([src_flat,
                                    jnp.zeros((pad,), jnp.int32)])
        dst_flat = jnp.concatenate([dst_flat, pad_dst])
    src = src_flat.reshape(NUM_SUBCORES, NBG, GB, K_EDGE)
    dst = dst_flat.reshape(NUM_SUBCORES, NBG, GB, K_EDGE)
    zeros = jnp.zeros((N, FC), jnp.float32)
    batch_r = batch.reshape(N // BR, 1, BR)
    batch_c = batch.reshape(N // BR, BR, 1)

    agg2 = _make_sc_agg(2)
    agg4 = _make_sc_agg(4)

    h = [x[:, c * FC:(c + 1) * FC] for c in range(F_IN // FC)]

    gin = params['gin']
    for l in range(2):
        lp = gin[l]
        a = agg2(*h, src, dst, zeros) if l == 0 else agg4(*h, src, dst, zeros)
        layer = _make_tc_layer(len(h), relu_out=True)
        h = list(layer(*h, *a, lp['W1'], lp['b1'].reshape(1, -1),
                       lp['W2'], lp['b2'].reshape(1, -1)))

    lp = gin[2]
    a = agg4(*h, src, dst, zeros)
    final = _make_tc_layer_final(len(h))
    node_out, _sums, _maxes, _counts, prop_out = final(
        *h, *a, lp['W1'], lp['b1'].reshape(1, -1),
        lp['W2'], lp['b2'].reshape(1, -1),
        params['nc_W'], params['nc_b'].reshape(1, -1),
        params['p1_W'], params['p1_b'].reshape(1, -1),
        params['p2_W'], params['p2_b'].reshape(1, -1),
        batch_r, batch_c)
    return node_out, prop_out
